# Initial kernel scaffold; baseline (speedup 1.0000x reference)
import jax
import jax.numpy as jnp
from jax.experimental import pallas as pl


def kernel(x, adj, W, att_src, att_dst, b):
    raise NotImplementedError("write your pallas kernel here")


# speedup vs baseline: 28.0380x; 28.0380x over previous
import functools

import jax
import jax.numpy as jnp
from jax import lax
from jax.experimental import pallas as pl
from jax.experimental.pallas import tpu as pltpu
from jax.experimental.pallas import tpu_sc as plsc

_N = 10000
_NPAD = 10240
_FIN = 128
_FOUT = 64
_E_RAW = 320000
_E_TOT = _E_RAW + _N
_CHUNK = 128
_ROWS_TOTAL = 2592
_EPAD = _ROWS_TOTAL * _CHUNK
_ROWS_PER_TILE = _ROWS_TOTAL // 32
_RPT = _ROWS_PER_TILE


def _pre_body(x_ref, w_ref, a2_ref, h_ref, ae_ref):
    h = jnp.dot(x_ref[...], w_ref[...], preferred_element_type=jnp.float32)
    h_ref[...] = h
    wa = jnp.dot(w_ref[...], a2_ref[...], preferred_element_type=jnp.float32)
    ae_ref[...] = jnp.dot(x_ref[...], wa, preferred_element_type=jnp.float32)


_pre_call = pl.pallas_call(
    _pre_body,
    out_shape=[
        jax.ShapeDtypeStruct((_N, _FOUT), jnp.float32),
        jax.ShapeDtypeStruct((_N, 2), jnp.float32),
    ],
)


_mesh = plsc.VectorSubcoreMesh(core_axis_name="c", subcore_axis_name="s")


@functools.partial(
    pl.kernel,
    out_type=[
        jax.ShapeDtypeStruct((2, _NPAD, _FOUT), jnp.float32),
        jax.ShapeDtypeStruct((2, _NPAD), jnp.float32),
    ],
    mesh=_mesh,
    scratch_types=[
        pltpu.VMEM((_RPT, _CHUNK), jnp.int32),
        pltpu.VMEM((_RPT, _CHUNK), jnp.int32),
        pltpu.VMEM((2 * _N,), jnp.float32),
        pltpu.VMEM((_RPT, _CHUNK), jnp.float32),
        pltpu.VMEM((_CHUNK, _FOUT), jnp.float32),
        pltpu.VMEM((_NPAD // 16,), jnp.float32),
        pltpu.VMEM_SHARED((_NPAD, _FOUT), jnp.float32),
        pltpu.VMEM_SHARED((_NPAD,), jnp.float32),
        pltpu.SemaphoreType.DMA,
    ],
    compiler_params=pltpu.CompilerParams(
        needs_layout_passes=False, use_tc_tiling_on_sc=False
    ),
)
def _edge_call(src_ref, dst_ref, ae_ref, h_ref,
               num_out, den_out,
               src_v, dst_v, ae_v, w_v, rows_v, z_v, num_acc, den_acc, sem):
    cid = lax.axis_index("c")
    sid = lax.axis_index("s")
    wid = sid * 2 + cid
    base_row = wid * _RPT

    pltpu.sync_copy(src_ref.at[wid], src_v)
    pltpu.sync_copy(dst_ref.at[wid], dst_v)
    pltpu.sync_copy(ae_ref, ae_v)

    rows_per_sub = _NPAD // 16
    o = sid * rows_per_sub

    def _zrow(r, _):
        for f in range(_FOUT // 16):
            rows_v[r, pl.ds(f * 16, 16)] = jnp.zeros((16,), jnp.float32)
        return 0

    lax.fori_loop(0, _CHUNK, _zrow, 0)

    def _zden(i, _):
        z_v[pl.ds(i * 16, 16)] = jnp.zeros((16,), jnp.float32)
        return 0

    lax.fori_loop(0, rows_per_sub // 16, _zden, 0)

    for blk in range(rows_per_sub // _CHUNK):
        pltpu.sync_copy(rows_v, num_acc.at[pl.ds(o + blk * _CHUNK, _CHUNK)])
    pltpu.sync_copy(z_v, den_acc.at[pl.ds(o, rows_per_sub)])

    lane = lax.iota(jnp.int32, 16)
    zeros16 = lax.iota(jnp.int32, 16) * 0
    ones16 = zeros16 + 1
    ebase = base_row * _CHUNK

    def _wrow(j, _):
        def _wvec(i, _):
            off = i * 16
            sv = src_v[j, pl.ds(off, 16)]
            dv = dst_v[j, pl.ds(off, 16)]
            e = (plsc.load_gather(ae_v, [sv + sv])
                 + plsc.load_gather(ae_v, [dv + dv + ones16]))
            e = jnp.maximum(e, 0.2 * e)
            w = jnp.exp(e)
            gid = ebase + j * _CHUNK + off + lane
            w = jnp.where(gid < _E_TOT, w, 0.0)
            w_v[j, pl.ds(off, 16)] = w
            return 0

        return lax.fori_loop(0, _CHUNK // 16, _wvec, 0)

    lax.fori_loop(0, _RPT, _wrow, 0)

    plsc.subcore_barrier()

    def _grow(j, _):
        pltpu.async_copy(h_ref.at[src_v.at[j]], rows_v, sem).wait()

        def _scale(i16, _):
            wvec = w_v[j, pl.ds(i16 * 16, 16)]
            for r in range(16):
                row = i16 * 16 + r
                wj = wvec[r]
                for f in range(_FOUT // 16):
                    sl = pl.ds(f * 16, 16)
                    rows_v[row, sl] = rows_v[row, sl] * wj
            return 0

        lax.fori_loop(0, _CHUNK // 16, _scale, 0)
        pltpu.sync_copy(rows_v, num_acc.at[dst_v.at[j]], add=True)
        pltpu.sync_copy(w_v.at[j], den_acc.at[dst_v.at[j]], add=True)
        return 0

    lax.fori_loop(0, _RPT, _grow, 0)

    plsc.subcore_barrier()

    pltpu.sync_copy(num_acc.at[pl.ds(o, rows_per_sub)],
                    num_out.at[cid, pl.ds(o, rows_per_sub)])
    pltpu.sync_copy(den_acc.at[pl.ds(o, rows_per_sub)],
                    den_out.at[cid, pl.ds(o, rows_per_sub)])


def _post_body(num_ref, den_ref, b_ref, o_ref):
    n = num_ref[0, : _N, :] + num_ref[1, : _N, :]
    d = den_ref[0, : _N] + den_ref[1, : _N]
    o_ref[...] = jnp.maximum(n / d[:, None] + b_ref[...][None, :], 0.0)


_post_call = pl.pallas_call(
    _post_body,
    out_shape=jax.ShapeDtypeStruct((_N, _FOUT), jnp.float32),
)


def kernel(x, adj, W, att_src, att_dst, b):
    src = adj[0]
    dst = adj[1]
    loop = jnp.arange(_N, dtype=src.dtype)
    pad = jnp.zeros((_EPAD - _E_TOT,), src.dtype)
    src2 = jnp.concatenate([src, loop, pad]).reshape(32, _RPT, _CHUNK)
    dst2 = jnp.concatenate([dst, loop, pad]).reshape(32, _RPT, _CHUNK)

    a2 = jnp.stack([att_src, att_dst], axis=1)
    h, ae = _pre_call(x, W, a2)
    num, den = _edge_call(src2, dst2, ae.reshape(-1), h)
    return _post_call(num, den, b)


# speedup vs baseline: 35.5768x; 1.2689x over previous
import functools

import jax
import jax.numpy as jnp
from jax import lax
from jax.experimental import pallas as pl
from jax.experimental.pallas import tpu as pltpu
from jax.experimental.pallas import tpu_sc as plsc

_N = 10000
_NPAD = 10240
_FIN = 128
_FOUT = 64
_E_RAW = 320000
_E_TOT = _E_RAW + _N
_CHUNK = 128
_ROWS_TOTAL = 2592
_EPAD = _ROWS_TOTAL * _CHUNK
_ROWS_PER_TILE = _ROWS_TOTAL // 32
_RPT = _ROWS_PER_TILE


def _pre_body(x_ref, w_ref, a2_ref, h_ref, ae_ref):
    h = jnp.dot(x_ref[...], w_ref[...], preferred_element_type=jnp.float32)
    h_ref[...] = h
    wa = jnp.dot(w_ref[...], a2_ref[...], preferred_element_type=jnp.float32)
    ae_ref[...] = jnp.dot(x_ref[...], wa, preferred_element_type=jnp.float32)


_pre_call = pl.pallas_call(
    _pre_body,
    out_shape=[
        jax.ShapeDtypeStruct((_N, _FOUT), jnp.float32),
        jax.ShapeDtypeStruct((_N, 2), jnp.float32),
    ],
)


_mesh = plsc.VectorSubcoreMesh(core_axis_name="c", subcore_axis_name="s")


@functools.partial(
    pl.kernel,
    out_type=[
        jax.ShapeDtypeStruct((2, _NPAD, _FOUT), jnp.float32),
        jax.ShapeDtypeStruct((2, _NPAD), jnp.float32),
    ],
    mesh=_mesh,
    scratch_types=[
        pltpu.VMEM((_RPT, _CHUNK), jnp.int32),
        pltpu.VMEM((_RPT, _CHUNK), jnp.int32),
        pltpu.VMEM((2 * _N,), jnp.float32),
        pltpu.VMEM((_RPT, _CHUNK), jnp.float32),
        pltpu.VMEM((_CHUNK, _FOUT), jnp.float32),
        pltpu.VMEM((_CHUNK, _FOUT), jnp.float32),
        pltpu.VMEM((_NPAD // 16,), jnp.float32),
        pltpu.VMEM_SHARED((_NPAD, _FOUT), jnp.float32),
        pltpu.VMEM_SHARED((_NPAD,), jnp.float32),
        pltpu.SemaphoreType.DMA,
        pltpu.SemaphoreType.DMA,
    ],
    compiler_params=pltpu.CompilerParams(
        needs_layout_passes=False, use_tc_tiling_on_sc=False
    ),
)
def _edge_call(src_ref, dst_ref, ae_ref, h_ref,
               num_out, den_out,
               src_v, dst_v, ae_v, w_v, rows_a, rows_b, z_v,
               num_acc, den_acc, sem_a, sem_b):
    cid = lax.axis_index("c")
    sid = lax.axis_index("s")
    wid = sid * 2 + cid
    base_row = wid * _RPT

    pltpu.sync_copy(src_ref.at[wid], src_v)
    pltpu.sync_copy(dst_ref.at[wid], dst_v)
    pltpu.sync_copy(ae_ref, ae_v)

    rows_per_sub = _NPAD // 16
    o = sid * rows_per_sub

    def _zrow(r, _):
        for f in range(_FOUT // 16):
            rows_a[r, pl.ds(f * 16, 16)] = jnp.zeros((16,), jnp.float32)
        return 0

    lax.fori_loop(0, _CHUNK, _zrow, 0)

    def _zden(i, _):
        z_v[pl.ds(i * 16, 16)] = jnp.zeros((16,), jnp.float32)
        return 0

    lax.fori_loop(0, rows_per_sub // 16, _zden, 0)

    for blk in range(rows_per_sub // _CHUNK):
        pltpu.sync_copy(rows_a, num_acc.at[pl.ds(o + blk * _CHUNK, _CHUNK)])
    pltpu.sync_copy(z_v, den_acc.at[pl.ds(o, rows_per_sub)])

    lane = lax.iota(jnp.int32, 16)
    zeros16 = lax.iota(jnp.int32, 16) * 0
    ones16 = zeros16 + 1
    ebase = base_row * _CHUNK

    def _wrow(j, _):
        def _wvec(i, _):
            off = i * 16
            sv = src_v[j, pl.ds(off, 16)]
            dv = dst_v[j, pl.ds(off, 16)]
            e = (plsc.load_gather(ae_v, [sv + sv])
                 + plsc.load_gather(ae_v, [dv + dv + ones16]))
            e = jnp.maximum(e, 0.2 * e)
            w = jnp.exp(e)
            gid = ebase + j * _CHUNK + off + lane
            w = jnp.where(gid < _E_TOT, w, 0.0)
            w_v[j, pl.ds(off, 16)] = w
            return 0

        return lax.fori_loop(0, _CHUNK // 16, _wvec, 0)

    lax.fori_loop(0, _RPT, _wrow, 0)

    plsc.subcore_barrier()

    def _scale(j, buf):
        def _s16(i16, _):
            wvec = w_v[j, pl.ds(i16 * 16, 16)]
            for r in range(16):
                row = i16 * 16 + r
                wj = wvec[r]
                for f in range(_FOUT // 16):
                    sl = pl.ds(f * 16, 16)
                    buf[row, sl] = buf[row, sl] * wj
            return 0

        lax.fori_loop(0, _CHUNK // 16, _s16, 0)

    def _process(j, buf):
        _scale(j, buf)
        pltpu.sync_copy(buf, num_acc.at[dst_v.at[j]], add=True)
        pltpu.sync_copy(w_v.at[j], den_acc.at[dst_v.at[j]], add=True)

    pltpu.async_copy(h_ref.at[src_v.at[0]], rows_a, sem_a)

    def _grow(p, _):
        j0 = 2 * p
        j1 = j0 + 1
        pltpu.make_async_copy(h_ref.at[src_v.at[j0]], rows_a, sem_a).wait()
        pltpu.async_copy(h_ref.at[src_v.at[j1]], rows_b, sem_b)
        _process(j0, rows_a)
        pltpu.make_async_copy(h_ref.at[src_v.at[j1]], rows_b, sem_b).wait()

        @pl.when(j0 + 2 < _RPT)
        def _():
            pltpu.async_copy(h_ref.at[src_v.at[j0 + 2]], rows_a, sem_a)

        _process(j1, rows_b)
        return 0

    lax.fori_loop(0, _RPT // 2, _grow, 0)
    jl = _RPT - 1
    pltpu.make_async_copy(h_ref.at[src_v.at[jl]], rows_a, sem_a).wait()
    _process(jl, rows_a)

    plsc.subcore_barrier()

    pltpu.sync_copy(num_acc.at[pl.ds(o, rows_per_sub)],
                    num_out.at[cid, pl.ds(o, rows_per_sub)])
    pltpu.sync_copy(den_acc.at[pl.ds(o, rows_per_sub)],
                    den_out.at[cid, pl.ds(o, rows_per_sub)])


def _post_body(num_ref, den_ref, b_ref, o_ref):
    n = num_ref[0, : _N, :] + num_ref[1, : _N, :]
    d = den_ref[0, : _N] + den_ref[1, : _N]
    o_ref[...] = jnp.maximum(n / d[:, None] + b_ref[...][None, :], 0.0)


_post_call = pl.pallas_call(
    _post_body,
    out_shape=jax.ShapeDtypeStruct((_N, _FOUT), jnp.float32),
)


def kernel(x, adj, W, att_src, att_dst, b):
    src = adj[0]
    dst = adj[1]
    loop = jnp.arange(_N, dtype=src.dtype)
    pad = jnp.zeros((_EPAD - _E_TOT,), src.dtype)
    src2 = jnp.concatenate([src, loop, pad]).reshape(32, _RPT, _CHUNK)
    dst2 = jnp.concatenate([dst, loop, pad]).reshape(32, _RPT, _CHUNK)

    a2 = jnp.stack([att_src, att_dst], axis=1)
    h, ae = _pre_call(x, W, a2)
    num, den = _edge_call(src2, dst2, ae.reshape(-1), h)
    return _post_call(num, den, b)


# speedup vs baseline: 39.5275x; 1.1110x over previous
import functools

import jax
import jax.numpy as jnp
from jax import lax
from jax.experimental import pallas as pl
from jax.experimental.pallas import tpu as pltpu
from jax.experimental.pallas import tpu_sc as plsc

_N = 10000
_NPAD = 10240
_FIN = 128
_FOUT = 64
_E_RAW = 320000
_E_TOT = _E_RAW + _N
_CHUNK = 128
_ROWS_TOTAL = 2592
_EPAD = _ROWS_TOTAL * _CHUNK
_ROWS_PER_TILE = _ROWS_TOTAL // 32
_RPT = _ROWS_PER_TILE


def _pre_body(x_ref, w_ref, a2_ref, h_ref, ae_ref):
    h = jnp.dot(x_ref[...], w_ref[...], preferred_element_type=jnp.float32)
    h_ref[...] = h
    wa = jnp.dot(w_ref[...], a2_ref[...], preferred_element_type=jnp.float32)
    ae_ref[...] = jnp.dot(x_ref[...], wa, preferred_element_type=jnp.float32)


_pre_call = pl.pallas_call(
    _pre_body,
    out_shape=[
        jax.ShapeDtypeStruct((_N, _FOUT), jnp.float32),
        jax.ShapeDtypeStruct((_N, 2), jnp.float32),
    ],
)


_mesh = plsc.VectorSubcoreMesh(core_axis_name="c", subcore_axis_name="s")


@functools.partial(
    pl.kernel,
    out_type=[
        jax.ShapeDtypeStruct((2, _NPAD, _FOUT), jnp.float32),
        jax.ShapeDtypeStruct((2, _NPAD), jnp.float32),
    ],
    mesh=_mesh,
    scratch_types=[
        pltpu.VMEM((_RPT, _CHUNK), jnp.int32),
        pltpu.VMEM((_RPT, _CHUNK), jnp.int32),
        pltpu.VMEM((2 * _N,), jnp.float32),
        pltpu.VMEM((_RPT, _CHUNK), jnp.float32),
        pltpu.VMEM((_CHUNK, _FOUT), jnp.float32),
        pltpu.VMEM((_CHUNK, _FOUT), jnp.float32),
        pltpu.VMEM((_CHUNK, _FOUT), jnp.float32),
        pltpu.VMEM((_CHUNK, _FOUT), jnp.float32),
        pltpu.VMEM((_NPAD // 16,), jnp.float32),
        pltpu.VMEM_SHARED((_NPAD, _FOUT), jnp.float32),
        pltpu.VMEM_SHARED((_NPAD,), jnp.float32),
        pltpu.SemaphoreType.DMA,
        pltpu.SemaphoreType.DMA,
        pltpu.SemaphoreType.DMA,
        pltpu.SemaphoreType.DMA,
        pltpu.SemaphoreType.DMA,
        pltpu.SemaphoreType.DMA,
        pltpu.SemaphoreType.DMA,
        pltpu.SemaphoreType.DMA,
        pltpu.SemaphoreType.DMA,
    ],
    compiler_params=pltpu.CompilerParams(
        needs_layout_passes=False, use_tc_tiling_on_sc=False
    ),
)
def _edge_call(src_ref, dst_ref, ae_ref, h_ref,
               num_out, den_out,
               src_v, dst_v, ae_v, w_v, rows0, rows1, rows2, rows3, z_v,
               num_acc, den_acc,
               g0, g1, g2, g3, s0, s1, s2, s3, dsem):
    cid = lax.axis_index("c")
    sid = lax.axis_index("s")
    wid = sid * 2 + cid
    base_row = wid * _RPT

    pltpu.sync_copy(src_ref.at[wid], src_v)
    pltpu.sync_copy(dst_ref.at[wid], dst_v)
    pltpu.sync_copy(ae_ref, ae_v)

    rows_per_sub = _NPAD // 16
    o = sid * rows_per_sub

    def _zrow(r, _):
        for f in range(_FOUT // 16):
            rows0[r, pl.ds(f * 16, 16)] = jnp.zeros((16,), jnp.float32)
        return 0

    lax.fori_loop(0, _CHUNK, _zrow, 0)

    def _zden(i, _):
        z_v[pl.ds(i * 16, 16)] = jnp.zeros((16,), jnp.float32)
        return 0

    lax.fori_loop(0, rows_per_sub // 16, _zden, 0)

    for blk in range(rows_per_sub // _CHUNK):
        pltpu.sync_copy(rows0, num_acc.at[pl.ds(o + blk * _CHUNK, _CHUNK)])
    pltpu.sync_copy(z_v, den_acc.at[pl.ds(o, rows_per_sub)])

    lane = lax.iota(jnp.int32, 16)
    zeros16 = lax.iota(jnp.int32, 16) * 0
    ones16 = zeros16 + 1
    ebase = base_row * _CHUNK

    def _wrow(j, _):
        def _wvec(i, _):
            off = i * 16
            sv = src_v[j, pl.ds(off, 16)]
            dv = dst_v[j, pl.ds(off, 16)]
            e = (plsc.load_gather(ae_v, [sv + sv])
                 + plsc.load_gather(ae_v, [dv + dv + ones16]))
            e = jnp.maximum(e, 0.2 * e)
            w = jnp.exp(e)
            gid = ebase + j * _CHUNK + off + lane
            w = jnp.where(gid < _E_TOT, w, 0.0)
            w_v[j, pl.ds(off, 16)] = w
            return 0

        return lax.fori_loop(0, _CHUNK // 16, _wvec, 0)

    lax.fori_loop(0, _RPT, _wrow, 0)

    plsc.subcore_barrier()

    bufs = (rows0, rows1, rows2, rows3)
    gsems = (g0, g1, g2, g3)
    ssems = (s0, s1, s2, s3)

    def _chunk_body(j, k):
        buf, gs, ss = bufs[k], gsems[k], ssems[k]
        k2 = (k + 2) % 4
        pltpu.make_async_copy(h_ref.at[src_v.at[j]], buf, gs).wait()

        @pl.when(j >= 2)
        def _():
            pltpu.make_async_copy(
                bufs[k2], num_acc.at[dst_v.at[j - 2]], ssems[k2]).wait()

        @pl.when(j + 2 < _RPT)
        def _():
            pltpu.async_copy(h_ref.at[src_v.at[j + 2]], bufs[k2], gsems[k2])

        def _s16(i16, _):
            wvec = w_v[j, pl.ds(i16 * 16, 16)]
            for r in range(16):
                row = i16 * 16 + r
                wj = wvec[r]
                for f in range(_FOUT // 16):
                    sl = pl.ds(f * 16, 16)
                    buf[row, sl] = buf[row, sl] * wj
            return 0

        lax.fori_loop(0, _CHUNK // 16, _s16, 0)
        pltpu.async_copy(buf, num_acc.at[dst_v.at[j]], ss, add=True)
        pltpu.async_copy(w_v.at[j], den_acc.at[dst_v.at[j]], dsem, add=True)

    pltpu.async_copy(h_ref.at[src_v.at[0]], bufs[0], gsems[0])
    pltpu.async_copy(h_ref.at[src_v.at[1]], bufs[1], gsems[1])

    def _grow(p, _):
        for k in range(4):
            _chunk_body(4 * p + k, k)
        return 0

    lax.fori_loop(0, _RPT // 4, _grow, 0)
    _chunk_body(_RPT - 1, (_RPT - 1) % 4)

    pltpu.make_async_copy(
        bufs[79 % 4], num_acc.at[dst_v.at[79]], ssems[79 % 4]).wait()
    pltpu.make_async_copy(
        bufs[80 % 4], num_acc.at[dst_v.at[80]], ssems[80 % 4]).wait()

    def _ddrain(j, _):
        pltpu.make_async_copy(w_v.at[j], den_acc.at[dst_v.at[j]], dsem).wait()
        return 0

    lax.fori_loop(0, _RPT, _ddrain, 0)

    plsc.subcore_barrier()

    pltpu.sync_copy(num_acc.at[pl.ds(o, rows_per_sub)],
                    num_out.at[cid, pl.ds(o, rows_per_sub)])
    pltpu.sync_copy(den_acc.at[pl.ds(o, rows_per_sub)],
                    den_out.at[cid, pl.ds(o, rows_per_sub)])


def _post_body(num_ref, den_ref, b_ref, o_ref):
    n = num_ref[0, : _N, :] + num_ref[1, : _N, :]
    d = den_ref[0, : _N] + den_ref[1, : _N]
    o_ref[...] = jnp.maximum(n / d[:, None] + b_ref[...][None, :], 0.0)


_post_call = pl.pallas_call(
    _post_body,
    out_shape=jax.ShapeDtypeStruct((_N, _FOUT), jnp.float32),
)


def kernel(x, adj, W, att_src, att_dst, b):
    src = adj[0]
    dst = adj[1]
    loop = jnp.arange(_N, dtype=src.dtype)
    pad = jnp.zeros((_EPAD - _E_TOT,), src.dtype)
    src2 = jnp.concatenate([src, loop, pad]).reshape(32, _RPT, _CHUNK)
    dst2 = jnp.concatenate([dst, loop, pad]).reshape(32, _RPT, _CHUNK)

    a2 = jnp.stack([att_src, att_dst], axis=1)
    h, ae = _pre_call(x, W, a2)
    num, den = _edge_call(src2, dst2, ae.reshape(-1), h)
    return _post_call(num, den, b)


# speedup vs baseline: 54.1246x; 1.3693x over previous
import functools

import jax
import jax.numpy as jnp
from jax import lax
from jax.experimental import pallas as pl
from jax.experimental.pallas import tpu as pltpu
from jax.experimental.pallas import tpu_sc as plsc

_N = 10000
_NPAD = 10240
_FIN = 128
_FOUT = 64
_E_RAW = 320000
_E_TOT = _E_RAW + _N
_CHUNK = 128
_ROWS_TOTAL = 2592
_EPAD = _ROWS_TOTAL * _CHUNK
_ROWS_PER_TILE = _ROWS_TOTAL // 32
_RPT = _ROWS_PER_TILE
_NSLOT = 4


def _pre_body(x_ref, w_ref, a2_ref, h_ref, ae_ref):
    h = jnp.dot(x_ref[...], w_ref[...], preferred_element_type=jnp.float32)
    h_ref[...] = h
    wa = jnp.dot(w_ref[...], a2_ref[...], preferred_element_type=jnp.float32)
    ae_ref[...] = jnp.dot(x_ref[...], wa, preferred_element_type=jnp.float32)


_pre_call = pl.pallas_call(
    _pre_body,
    out_shape=[
        jax.ShapeDtypeStruct((_N, _FOUT), jnp.float32),
        jax.ShapeDtypeStruct((_N, 2), jnp.float32),
    ],
)


_mesh = plsc.VectorSubcoreMesh(core_axis_name="c", subcore_axis_name="s")


@functools.partial(
    pl.kernel,
    out_type=[
        jax.ShapeDtypeStruct((2, _NPAD, _FOUT), jnp.float32),
        jax.ShapeDtypeStruct((2, _NPAD), jnp.float32),
    ],
    mesh=_mesh,
    scratch_types=(
        [
            pltpu.VMEM((_RPT, _CHUNK), jnp.int32),
            pltpu.VMEM((_RPT, _CHUNK), jnp.int32),
            pltpu.VMEM((2 * _N,), jnp.float32),
            pltpu.VMEM((_RPT, _CHUNK), jnp.float32),
            pltpu.VMEM((_NPAD // 16,), jnp.float32),
        ]
        + [pltpu.VMEM((_CHUNK, _FOUT), jnp.float32)] * _NSLOT
        + [
            pltpu.VMEM_SHARED((_NPAD, _FOUT), jnp.float32),
            pltpu.VMEM_SHARED((_NPAD,), jnp.float32),
        ]
        + [pltpu.SemaphoreType.DMA] * (2 * _NSLOT + 1)
    ),
    compiler_params=pltpu.CompilerParams(
        needs_layout_passes=False, use_tc_tiling_on_sc=False
    ),
)
def _edge_call(src_ref, dst_ref, ae_ref, h_ref,
               num_out, den_out,
               src_v, dst_v, ae_v, w_v, z_v,
               r0, r1, r2, r3,
               num_acc, den_acc,
               g0, g1, g2, g3, s0, s1, s2, s3, dsem):
    cid = lax.axis_index("c")
    sid = lax.axis_index("s")
    wid = sid * 2 + cid
    base_row = wid * _RPT

    pltpu.sync_copy(src_ref.at[wid], src_v)
    pltpu.sync_copy(dst_ref.at[wid], dst_v)
    pltpu.sync_copy(ae_ref, ae_v)

    rows_per_sub = _NPAD // 16
    o = sid * rows_per_sub

    def _zrow(r, _):
        for f in range(_FOUT // 16):
            r0[r, pl.ds(f * 16, 16)] = jnp.zeros((16,), jnp.float32)
        return 0

    lax.fori_loop(0, _CHUNK, _zrow, 0)

    def _zden(i, _):
        z_v[pl.ds(i * 16, 16)] = jnp.zeros((16,), jnp.float32)
        return 0

    lax.fori_loop(0, rows_per_sub // 16, _zden, 0)

    for blk in range(rows_per_sub // _CHUNK):
        pltpu.sync_copy(r0, num_acc.at[pl.ds(o + blk * _CHUNK, _CHUNK)])
    pltpu.sync_copy(z_v, den_acc.at[pl.ds(o, rows_per_sub)])

    lane = lax.iota(jnp.int32, 16)
    zeros16 = lane * 0
    ones16 = zeros16 + 1
    ebase = base_row * _CHUNK

    def _wchunk(jj):
        def _wvec(i, _):
            off = i * 16
            sv = src_v[jj, pl.ds(off, 16)]
            dv = dst_v[jj, pl.ds(off, 16)]
            e = (plsc.load_gather(ae_v, [sv + sv])
                 + plsc.load_gather(ae_v, [dv + dv + ones16]))
            e = jnp.maximum(e, 0.2 * e)
            w = jnp.exp(e)
            gid = ebase + jj * _CHUNK + off + lane
            w = jnp.where(gid < _E_TOT, w, 0.0)
            w_v[jj, pl.ds(off, 16)] = w
            return 0

        lax.fori_loop(0, _CHUNK // 16, _wvec, 0)

    plsc.subcore_barrier()

    bufs = (r0, r1, r2, r3)
    gsems = (g0, g1, g2, g3)
    ssems = (s0, s1, s2, s3)
    _PF = _NSLOT - 2

    def _chunk_body(j, k):
        buf, gs, ss = bufs[k], gsems[k], ssems[k]
        kp = (k + _PF) % _NSLOT
        pltpu.make_async_copy(h_ref.at[src_v.at[j]], buf, gs).wait()

        @pl.when(j >= 2)
        def _():
            pltpu.make_async_copy(
                bufs[kp], num_acc.at[dst_v.at[j - 2]], ssems[kp]).wait()

        @pl.when(j + _PF < _RPT)
        def _():
            pltpu.async_copy(h_ref.at[src_v.at[j + _PF]], bufs[kp], gsems[kp])
            _wchunk(j + _PF)

        jv = zeros16 + j

        def _s(r, rvec):
            wb = plsc.load_gather(w_v, [jv, rvec])
            for f in range(_FOUT // 16):
                sl = pl.ds(f * 16, 16)
                buf[r, sl] = buf[r, sl] * wb
            return rvec + 1

        lax.fori_loop(0, _CHUNK, _s, zeros16, unroll=4)
        pltpu.async_copy(buf, num_acc.at[dst_v.at[j]], ss, add=True)
        pltpu.async_copy(w_v.at[j], den_acc.at[dst_v.at[j]], dsem, add=True)

    for jj in range(_PF):
        pltpu.async_copy(h_ref.at[src_v.at[jj]], bufs[jj], gsems[jj])
    for jj in range(_PF):
        _wchunk(jj)

    def _grow(p, _):
        for k in range(_NSLOT):
            _chunk_body(_NSLOT * p + k, k)
        return 0

    lax.fori_loop(0, _RPT // _NSLOT, _grow, 0)
    for j in range(_RPT - _RPT % _NSLOT, _RPT):
        _chunk_body(j, j % _NSLOT)

    for j in range(_RPT - 2, _RPT):
        pltpu.make_async_copy(
            bufs[j % _NSLOT], num_acc.at[dst_v.at[j]], ssems[j % _NSLOT]).wait()

    def _ddrain(j, _):
        pltpu.make_async_copy(w_v.at[j], den_acc.at[dst_v.at[j]], dsem).wait()
        return 0

    lax.fori_loop(0, _RPT, _ddrain, 0)

    plsc.subcore_barrier()

    pltpu.sync_copy(num_acc.at[pl.ds(o, rows_per_sub)],
                    num_out.at[cid, pl.ds(o, rows_per_sub)])
    pltpu.sync_copy(den_acc.at[pl.ds(o, rows_per_sub)],
                    den_out.at[cid, pl.ds(o, rows_per_sub)])


def _post_body(num_ref, den_ref, b_ref, o_ref):
    n = num_ref[0, : _N, :] + num_ref[1, : _N, :]
    d = den_ref[0, : _N] + den_ref[1, : _N]
    o_ref[...] = jnp.maximum(n / d[:, None] + b_ref[...][None, :], 0.0)


_post_call = pl.pallas_call(
    _post_body,
    out_shape=jax.ShapeDtypeStruct((_N, _FOUT), jnp.float32),
)


def kernel(x, adj, W, att_src, att_dst, b):
    src = adj[0]
    dst = adj[1]
    loop = jnp.arange(_N, dtype=src.dtype)
    pad = jnp.zeros((_EPAD - _E_TOT,), src.dtype)
    src2 = jnp.concatenate([src, loop, pad]).reshape(32, _RPT, _CHUNK)
    dst2 = jnp.concatenate([dst, loop, pad]).reshape(32, _RPT, _CHUNK)

    a2 = jnp.stack([att_src, att_dst], axis=1)
    h, ae = _pre_call(x, W, a2)
    num, den = _edge_call(src2, dst2, ae.reshape(-1), h)
    return _post_call(num, den, b)
